# baseline (speedup 1.0000x reference)
import functools

import jax
import jax.numpy as jnp
from jax import lax
from jax.experimental import pallas as pl
from jax.experimental.pallas import tpu as pltpu
from jax.experimental.pallas import tpu_sc as plsc

VOCAB = 1000000
DIM = 64
SEQ = 16384

NC = 2
NS = 16
NW = NC * NS
N_FULL_COLS = VOCAB // 128
N_COLS = N_FULL_COLS + 1
B_PER_W = SEQ // NW
RB = 6
SR = 8
N_VREG = SEQ // 16
DUMP = SEQ
OUTP_ROWS = SEQ + 16


def _scan_body(
    idx_hbm,
    wt_hbm,
    tail_hbm,
    outp_hbm,
    ids_v,
    own_k,
    own_p,
    colbuf,
    stg,
    ribuf,
    csem,
    ssem,
):
    wid = lax.axis_index("s") * NC + lax.axis_index("c")
    n_cols = (N_COLS // NW) + jnp.where(wid < N_COLS % NW, 1, 0)
    pltpu.sync_copy(idx_hbm, ids_v)
    iota16 = lax.iota(jnp.int32, 16)

    def pre(vg, nacc):
        iv = ids_v[pl.ds(vg * 16, 16)]
        col = iv >> 7
        m = (col & (NW - 1)) == wid
        c = plsc.all_reduce_population_count(m)[0]

        @pl.when(c > 0)
        def _():
            pos = nacc + plsc.cumsum(m.astype(jnp.int32)) - 1
            plsc.store_scatter(own_k, [pos], col >> 5, mask=m)
            packed = ((iv & 127) << 14) + (vg * 16 + iota16)
            plsc.store_scatter(own_p, [pos], packed, mask=m)

        return nacc + c

    nacc = lax.fori_loop(0, N_VREG, pre, 0)
    n_own_vregs = (nacc + 15) >> 4

    def fire_col(k, slot):
        col = wid + k * NW

        @pl.when(col < N_FULL_COLS)
        def _():
            c0 = pl.multiple_of(col * 128, 128)
            pltpu.async_copy(
                wt_hbm.at[:, pl.ds(c0, 128)], colbuf.at[slot], csem.at[slot]
            )

        @pl.when(col >= N_FULL_COLS)
        def _():
            pltpu.async_copy(tail_hbm, colbuf.at[slot], csem.at[slot])

    for s in range(RB):
        fire_col(s, s)

    def col_body(k, mcnt):
        slot = lax.rem(k, RB)
        pltpu.make_async_copy(
            wt_hbm.at[:, pl.ds(0, 128)], colbuf.at[slot], csem.at[slot]
        ).wait()

        def vreg_body(vg, mcnt):
            kv = own_k[pl.ds(vg * 16, 16)]
            m = (kv == k) & ((vg * 16 + iota16) < nacc)
            cnt = plsc.all_reduce_population_count(m)[0]

            @pl.when(cnt > 0)
            def _():
                r = mcnt & 15
                q = mcnt >> 4
                e = q + jnp.where(r > 0, 1, 0)
                entered = (r == 0) | ((r + cnt) >= 17)

                @pl.when(entered & (e >= SR))
                def _():
                    pltpu.make_async_copy(
                        stg.at[0],
                        outp_hbm.at[pl.ds(0, 16), :],
                        ssem.at[e & (SR - 1)],
                    ).wait()

                pv = own_p[pl.ds(vg * 16, 16)]
                lanes = pv >> 14
                bv = pv & (SEQ - 1)
                pos = plsc.cumsum(m.astype(jnp.int32)) - 1
                posg = mcnt + pos
                sv = (posg >> 4) & (SR - 1)
                rv = posg & 15
                plsc.store_scatter(ribuf, [sv, rv], bv, mask=m)

                def one_match(j, mm):
                    ffs = plsc.all_reduce_ffs(mm)
                    lane = jnp.sum(jnp.where(iota16 == ffs, lanes, 0))
                    pg = mcnt + j
                    svj = jnp.full((16,), (pg >> 4) & (SR - 1), jnp.int32)
                    rvj = jnp.full((16,), pg & 15, jnp.int32)
                    for cb in range(DIM // 16):
                        cvec = iota16 + cb * 16
                        vals = plsc.load_gather(
                            colbuf,
                            [
                                jnp.full((16,), slot, jnp.int32),
                                cvec,
                                jnp.full((16,), lane, jnp.int32),
                            ],
                        )
                        plsc.store_scatter(stg, [svj, rvj, cvec], vals)
                    return mm & (iota16 != ffs)

                lax.fori_loop(0, cnt, one_match, m)

                @pl.when((r + cnt) >= 16)
                def _():
                    qs = q & (SR - 1)
                    pltpu.async_copy(
                        stg.at[qs],
                        outp_hbm.at[ribuf.at[qs]],
                        ssem.at[qs],
                    )

            return mcnt + cnt

        mcnt = lax.fori_loop(0, n_own_vregs, vreg_body, mcnt)

        @pl.when(k + RB < n_cols)
        def _():
            fire_col(k + RB, slot)

        return mcnt

    mcnt = lax.fori_loop(0, n_cols, col_body, 0)

    r_f = mcnt & 15
    q_f = mcnt >> 4

    @pl.when(r_f > 0)
    def _():
        qs = q_f & (SR - 1)
        plsc.store_scatter(
            ribuf,
            [jnp.full((16,), qs, jnp.int32), iota16],
            jnp.full((16,), DUMP, jnp.int32) + iota16,
            mask=iota16 >= r_f,
        )
        pltpu.async_copy(
            stg.at[qs], outp_hbm.at[ribuf.at[qs]], ssem.at[qs]
        )

    total = q_f + jnp.where(r_f > 0, 1, 0)

    def drain(s, carry):
        @pl.when(s < jnp.minimum(total, SR))
        def _():
            pltpu.make_async_copy(
                stg.at[0], outp_hbm.at[pl.ds(0, 16), :], ssem.at[s]
            ).wait()

        return carry

    lax.fori_loop(0, SR, drain, 0)


def _tr_body(outp_hbm, out_hbm, inbuf, trows, wsem):
    wid = lax.axis_index("s") * NC + lax.axis_index("c")
    b0 = pl.multiple_of(wid * B_PER_W, B_PER_W)
    pltpu.sync_copy(outp_hbm.at[pl.ds(b0, B_PER_W), :], inbuf)
    iota16 = lax.iota(jnp.int32, 16)

    def tr(t, carry):
        c = t >> 5
        bb = t & 31
        vals = plsc.load_gather(
            inbuf, [iota16 + bb * 16, jnp.full((16,), c, jnp.int32)]
        )
        trows[pl.ds(c * B_PER_W + bb * 16, 16)] = vals
        return carry

    lax.fori_loop(0, DIM * (B_PER_W // 16), tr, 0)

    def put(c, carry):
        dst = pl.multiple_of(c * SEQ + b0, B_PER_W)
        pltpu.async_copy(
            trows.at[pl.ds(c * B_PER_W, B_PER_W)],
            out_hbm.at[pl.ds(dst, B_PER_W)],
            wsem,
        )
        return carry

    lax.fori_loop(0, DIM, put, 0)
    pltpu.make_async_copy(
        trows, out_hbm.at[pl.ds(0, DIM * B_PER_W)], wsem
    ).wait()


def _scan(idx, wt, tail):
    mesh = plsc.VectorSubcoreMesh(core_axis_name="c", subcore_axis_name="s")
    f = functools.partial(
        pl.kernel,
        mesh=mesh,
        out_type=jax.ShapeDtypeStruct((OUTP_ROWS, 128), jnp.float32),
        scratch_types=[
            pltpu.VMEM((SEQ,), jnp.int32),
            pltpu.VMEM((SEQ,), jnp.int32),
            pltpu.VMEM((SEQ,), jnp.int32),
            pltpu.VMEM((RB, DIM, 128), jnp.float32),
            pltpu.VMEM((SR, 16, 128), jnp.float32),
            pltpu.VMEM((SR, 16), jnp.int32),
            pltpu.SemaphoreType.DMA((RB,)),
            pltpu.SemaphoreType.DMA((SR,)),
        ],
        compiler_params=pltpu.CompilerParams(needs_layout_passes=False),
    )(_scan_body)
    return f(idx, wt, tail)


def _transpose(outp):
    mesh = plsc.VectorSubcoreMesh(core_axis_name="c", subcore_axis_name="s")
    f = functools.partial(
        pl.kernel,
        mesh=mesh,
        out_type=jax.ShapeDtypeStruct((DIM * SEQ,), jnp.float32),
        scratch_types=[
            pltpu.VMEM((B_PER_W, 128), jnp.float32),
            pltpu.VMEM((DIM * B_PER_W,), jnp.float32),
            pltpu.SemaphoreType.DMA,
        ],
        compiler_params=pltpu.CompilerParams(needs_layout_passes=False),
    )(_tr_body)
    return f(outp)


def kernel(input_ids, weight):
    idx = input_ids.reshape(SEQ).astype(jnp.int32)
    wt = weight.T
    tail = jnp.pad(wt[:, N_FULL_COLS * 128 :], ((0, 0), (0, 128 - VOCAB % 128)))
    outp = _scan(idx, wt, tail)
    flat = _transpose(outp)
    return flat.reshape(DIM, SEQ).T.reshape(SEQ, DIM, 1, 1)


# speedup vs baseline: 1.0149x; 1.0149x over previous
import functools

import jax
import jax.numpy as jnp
from jax import lax
from jax.experimental import pallas as pl
from jax.experimental.pallas import tpu as pltpu
from jax.experimental.pallas import tpu_sc as plsc

VOCAB = 1000000
DIM = 64
SEQ = 16384

NC = 2
NS = 16
NW = NC * NS
N_FULL_COLS = VOCAB // 128
N_COLS = N_FULL_COLS + 1
B_PER_W = SEQ // NW
RB = 6
SR = 8
N_VREG = SEQ // 16
DUMP = SEQ
OUTP_ROWS = SEQ + 16


def _scan_body(
    idx_hbm,
    wt_hbm,
    tail_hbm,
    outp_hbm,
    ids_v,
    own_k,
    own_p,
    colbuf,
    stg,
    ribuf,
    csem,
    ssem,
):
    wid = lax.axis_index("s") * NC + lax.axis_index("c")
    n_cols = (N_COLS // NW) + jnp.where(wid < N_COLS % NW, 1, 0)
    pltpu.sync_copy(idx_hbm, ids_v)
    iota16 = lax.iota(jnp.int32, 16)

    def pre(vg, nacc):
        iv = ids_v[pl.ds(vg * 16, 16)]
        col = iv >> 7
        m = (col & (NW - 1)) == wid
        c = plsc.all_reduce_population_count(m)[0]

        @pl.when(c > 0)
        def _():
            pos = nacc + plsc.cumsum(m.astype(jnp.int32)) - 1
            plsc.store_scatter(own_k, [pos], col >> 5, mask=m)
            packed = ((iv & 127) << 14) + (vg * 16 + iota16)
            plsc.store_scatter(own_p, [pos], packed, mask=m)

        return nacc + c

    nacc = lax.fori_loop(0, N_VREG, pre, 0)
    n_own_vregs = (nacc + 15) >> 4
    plsc.store_scatter(
        own_k,
        [jnp.minimum(nacc + iota16, SEQ - 1)],
        jnp.full((16,), -1, jnp.int32),
        mask=(nacc + iota16) < (n_own_vregs << 4),
    )

    def fire_col(k, slot):
        col = wid + k * NW

        @pl.when(col < N_FULL_COLS)
        def _():
            c0 = pl.multiple_of(col * 128, 128)
            pltpu.async_copy(
                wt_hbm.at[:, pl.ds(c0, 128)], colbuf.at[slot], csem.at[slot]
            )

        @pl.when(col >= N_FULL_COLS)
        def _():
            pltpu.async_copy(tail_hbm, colbuf.at[slot], csem.at[slot])

    for s in range(RB):
        fire_col(s, s)

    def col_body(k, mcnt):
        slot = lax.rem(k, RB)
        pltpu.make_async_copy(
            wt_hbm.at[:, pl.ds(0, 128)], colbuf.at[slot], csem.at[slot]
        ).wait()

        def vreg_body(vg, mcnt):
            kv = own_k[pl.ds(vg * 16, 16)]
            m = kv == k
            cnt = plsc.all_reduce_population_count(m)[0]

            @pl.when(cnt > 0)
            def _():
                r = mcnt & 15
                q = mcnt >> 4
                e = q + jnp.where(r > 0, 1, 0)
                entered = (r == 0) | ((r + cnt) >= 17)

                @pl.when(entered & (e >= SR))
                def _():
                    pltpu.make_async_copy(
                        stg.at[0],
                        outp_hbm.at[pl.ds(0, 16), :],
                        ssem.at[e & (SR - 1)],
                    ).wait()

                pv = own_p[pl.ds(vg * 16, 16)]
                lanes = pv >> 14
                bv = pv & (SEQ - 1)
                pos = plsc.cumsum(m.astype(jnp.int32)) - 1
                posg = mcnt + pos
                sv = (posg >> 4) & (SR - 1)
                rv = posg & 15
                plsc.store_scatter(ribuf, [sv, rv], bv, mask=m)

                def one_match(j, mm):
                    ffs = plsc.all_reduce_ffs(mm)
                    lane = jnp.sum(jnp.where(iota16 == ffs, lanes, 0))
                    pg = mcnt + j
                    svj = jnp.full((16,), (pg >> 4) & (SR - 1), jnp.int32)
                    rvj = jnp.full((16,), pg & 15, jnp.int32)
                    for cb in range(DIM // 16):
                        cvec = iota16 + cb * 16
                        vals = plsc.load_gather(
                            colbuf,
                            [
                                jnp.full((16,), slot, jnp.int32),
                                cvec,
                                jnp.full((16,), lane, jnp.int32),
                            ],
                        )
                        plsc.store_scatter(stg, [svj, rvj, cvec], vals)
                    return mm & (iota16 != ffs)

                lax.fori_loop(0, cnt, one_match, m)

                @pl.when((r + cnt) >= 16)
                def _():
                    qs = q & (SR - 1)
                    pltpu.async_copy(
                        stg.at[qs],
                        outp_hbm.at[ribuf.at[qs]],
                        ssem.at[qs],
                    )

            return mcnt + cnt

        mcnt = lax.fori_loop(0, n_own_vregs, vreg_body, mcnt)

        @pl.when(k + RB < n_cols)
        def _():
            fire_col(k + RB, slot)

        return mcnt

    mcnt = lax.fori_loop(0, n_cols, col_body, 0)

    r_f = mcnt & 15
    q_f = mcnt >> 4

    @pl.when(r_f > 0)
    def _():
        qs = q_f & (SR - 1)
        plsc.store_scatter(
            ribuf,
            [jnp.full((16,), qs, jnp.int32), iota16],
            jnp.full((16,), DUMP, jnp.int32) + iota16,
            mask=iota16 >= r_f,
        )
        pltpu.async_copy(
            stg.at[qs], outp_hbm.at[ribuf.at[qs]], ssem.at[qs]
        )

    total = q_f + jnp.where(r_f > 0, 1, 0)

    def drain(s, carry):
        @pl.when(s < jnp.minimum(total, SR))
        def _():
            pltpu.make_async_copy(
                stg.at[0], outp_hbm.at[pl.ds(0, 16), :], ssem.at[s]
            ).wait()

        return carry

    lax.fori_loop(0, SR, drain, 0)


def _tr_body(outp_hbm, out_hbm, inbuf, trows, wsem):
    wid = lax.axis_index("s") * NC + lax.axis_index("c")
    b0 = pl.multiple_of(wid * B_PER_W, B_PER_W)
    pltpu.sync_copy(outp_hbm.at[pl.ds(b0, B_PER_W), :], inbuf)
    iota16 = lax.iota(jnp.int32, 16)

    def tr(t, carry):
        c = t >> 5
        bb = t & 31
        vals = plsc.load_gather(
            inbuf, [iota16 + bb * 16, jnp.full((16,), c, jnp.int32)]
        )
        trows[pl.ds(c * B_PER_W + bb * 16, 16)] = vals
        return carry

    lax.fori_loop(0, DIM * (B_PER_W // 16), tr, 0)

    def put(c, carry):
        dst = pl.multiple_of(c * SEQ + b0, B_PER_W)
        pltpu.async_copy(
            trows.at[pl.ds(c * B_PER_W, B_PER_W)],
            out_hbm.at[pl.ds(dst, B_PER_W)],
            wsem,
        )
        return carry

    lax.fori_loop(0, DIM, put, 0)
    pltpu.make_async_copy(
        trows, out_hbm.at[pl.ds(0, DIM * B_PER_W)], wsem
    ).wait()


def _scan(idx, wt, tail):
    mesh = plsc.VectorSubcoreMesh(core_axis_name="c", subcore_axis_name="s")
    f = functools.partial(
        pl.kernel,
        mesh=mesh,
        out_type=jax.ShapeDtypeStruct((OUTP_ROWS, 128), jnp.float32),
        scratch_types=[
            pltpu.VMEM((SEQ,), jnp.int32),
            pltpu.VMEM((SEQ,), jnp.int32),
            pltpu.VMEM((SEQ,), jnp.int32),
            pltpu.VMEM((RB, DIM, 128), jnp.float32),
            pltpu.VMEM((SR, 16, 128), jnp.float32),
            pltpu.VMEM((SR, 16), jnp.int32),
            pltpu.SemaphoreType.DMA((RB,)),
            pltpu.SemaphoreType.DMA((SR,)),
        ],
        compiler_params=pltpu.CompilerParams(needs_layout_passes=False),
    )(_scan_body)
    return f(idx, wt, tail)


def _transpose(outp):
    mesh = plsc.VectorSubcoreMesh(core_axis_name="c", subcore_axis_name="s")
    f = functools.partial(
        pl.kernel,
        mesh=mesh,
        out_type=jax.ShapeDtypeStruct((DIM * SEQ,), jnp.float32),
        scratch_types=[
            pltpu.VMEM((B_PER_W, 128), jnp.float32),
            pltpu.VMEM((DIM * B_PER_W,), jnp.float32),
            pltpu.SemaphoreType.DMA,
        ],
        compiler_params=pltpu.CompilerParams(needs_layout_passes=False),
    )(_tr_body)
    return f(outp)


def kernel(input_ids, weight):
    idx = input_ids.reshape(SEQ).astype(jnp.int32)
    wt = weight.T
    tail = jnp.pad(wt[:, N_FULL_COLS * 128 :], ((0, 0), (0, 128 - VOCAB % 128)))
    outp = _scan(idx, wt, tail)
    flat = _transpose(outp)
    return flat.reshape(DIM, SEQ).T.reshape(SEQ, DIM, 1, 1)
